# Initial kernel scaffold; baseline (speedup 1.0000x reference)
import jax
import jax.numpy as jnp
from jax.experimental import pallas as pl


def kernel(dst_nodes, src_nodes, dst_feat, src_feat, edge_feat_nbr, delta_ts_nbr, root_ts, root_edge_feat, memory, memory_ts, params):
    raise NotImplementedError("write your pallas kernel here")


# speedup vs baseline: 1.0810x; 1.0810x over previous
import dataclasses
import functools
import math

import jax
import jax.numpy as jnp
from jax import lax
from jax.experimental import pallas as pl
from jax.experimental.pallas import tpu as pltpu
from jax.experimental.pallas import tpu_sc as plsc

N_NODES = 100000
B = 4096
GROUP = 3 * B
FANOUT = 16
D_NODE = 128
D_EDGE = 16
D_TIME = 100
D_EMBED = 128
N_HEADS = 2
DH = D_EMBED // N_HEADS

GATHER_WIN = 128
M_BLK = 64
R_BLK = M_BLK * FANOUT
G_BLK = 512
P_BLK = 512
N_WORKERS = 32
TS_ROWS = (N_NODES + 127) // 128
TS_SLICE = 3136
TS_PAD = N_WORKERS * TS_SLICE


def _sc_mesh():
  return plsc.VectorSubcoreMesh(core_axis_name="core", subcore_axis_name="subcore")


def _sc_gather(memory, idx_all, ts_tab, ts_row_idx):
  k_rows = idx_all.shape[0]
  k_ts = ts_row_idx.shape[0]
  idx2 = idx_all.reshape(1, k_rows)
  tsr2 = ts_row_idx.reshape(1, k_ts)

  @pl.kernel(
      out_type=(
          jax.ShapeDtypeStruct((k_rows, D_EMBED), jnp.float32),
          jax.ShapeDtypeStruct((k_ts, 128), jnp.float32),
      ),
      mesh=_sc_mesh(),
  )
  def gather_kernel(mem_hbm, idx_hbm, ts_hbm, tsr_hbm, rows_hbm, tsrows_hbm):
    def row_body(i_vmem, o_vmem):
      pltpu.sync_copy(mem_hbm.at[i_vmem.at[0]], o_vmem)

    pltpu.emit_pipeline(
        row_body,
        grid=(k_rows // GATHER_WIN,),
        in_specs=[pl.BlockSpec((1, GATHER_WIN), lambda i: (0, i))],
        out_specs=[pl.BlockSpec((GATHER_WIN, D_EMBED), lambda i: (i, 0))],
        core_axis_name=("core", "subcore"),
        dimension_semantics=(pltpu.PARALLEL,),
    )(idx_hbm, rows_hbm)

    def ts_body(i_vmem, o_vmem):
      pltpu.sync_copy(ts_hbm.at[i_vmem.at[0]], o_vmem)

    pltpu.emit_pipeline(
        ts_body,
        grid=(k_ts // GATHER_WIN,),
        in_specs=[pl.BlockSpec((1, GATHER_WIN), lambda i: (0, i))],
        out_specs=[pl.BlockSpec((GATHER_WIN, 128), lambda i: (i, 0))],
        core_axis_name=("core", "subcore"),
        dimension_semantics=(pltpu.PARALLEL,),
    )(tsr_hbm, tsrows_hbm)

  return gather_kernel(memory, idx2, ts_tab, tsr2)


def _sc_scatter(new_mem, win_src, dst_idx, mail_ts, ts_in, mem_ref):
  n_upd = dst_idx.shape[0]
  win2 = win_src.reshape(1, n_upd)
  dst2 = dst_idx.reshape(1, n_upd)

  cp = pltpu.CompilerParams()
  if "needs_layout_passes" in pltpu.CompilerParams.__dataclass_fields__:
    cp = dataclasses.replace(cp, needs_layout_passes=False)

  @pl.kernel(
      out_type=jax.ShapeDtypeStruct((TS_PAD,), jnp.float32),
      mesh=_sc_mesh(),
      compiler_params=cp,
      scratch_types=[
          pltpu.VMEM((GATHER_WIN, D_EMBED), jnp.float32),
          pltpu.VMEM((TS_SLICE,), jnp.float32),
          pltpu.VMEM((n_upd,), jnp.int32),
          pltpu.VMEM((n_upd,), jnp.int32),
          pltpu.VMEM((n_upd,), jnp.float32),
      ],
  )
  def scatter_kernel(nm_hbm, win_hbm, dst_hbm, mts_hbm, tsin_hbm, mem_hbm,
                     tsout_hbm, rows_vmem, tsslice_v, upd_v, win_v, mail_v):
    def body(w_vmem, d_vmem):
      pltpu.sync_copy(nm_hbm.at[w_vmem.at[0]], rows_vmem)
      pltpu.sync_copy(rows_vmem, mem_hbm.at[d_vmem.at[0]])

    pltpu.emit_pipeline(
        body,
        grid=(n_upd // GATHER_WIN,),
        in_specs=[
            pl.BlockSpec((1, GATHER_WIN), lambda i: (0, i)),
            pl.BlockSpec((1, GATHER_WIN), lambda i: (0, i)),
        ],
        out_specs=[],
        core_axis_name=("core", "subcore"),
        dimension_semantics=(pltpu.PARALLEL,),
    )(win_hbm, dst_hbm)

    wid = lax.axis_index("core") * 16 + lax.axis_index("subcore")
    base = wid * TS_SLICE
    pltpu.sync_copy(tsin_hbm.at[pl.ds(base, TS_SLICE)], tsslice_v)
    pltpu.sync_copy(dst_hbm.at[0], upd_v)
    pltpu.sync_copy(win_hbm.at[0], win_v)
    pltpu.sync_copy(mts_hbm, mail_v)

    @pl.loop(0, n_upd // 16)
    def _(c):
      iv = upd_v[pl.ds(c * 16, 16)]
      wv = win_v[pl.ds(c * 16, 16)]
      vals = plsc.load_gather(mail_v, [wv])
      mask = (iv >= base) & (iv < base + TS_SLICE)
      plsc.store_scatter(tsslice_v, [iv - base], vals, mask=mask)

    pltpu.sync_copy(tsslice_v, tsout_hbm.at[pl.ds(base, TS_SLICE)])

  return scatter_kernel(new_mem, win2, dst2, mail_ts, ts_in, mem_ref)


def _attn_body(gdst_ref, dstf_ref, gsrc_ref, srcf_ref, edge_ref, delta_ref,
               wt_ref, bt_ref, wqn_ref, wqt_ref, bq_ref,
               wkn_ref, wke_ref, wkt_ref, bk_ref,
               wvn_ref, wve_ref, wvt_ref, bv_ref,
               wo_ref, bo_ref, h_ref):
  f32 = jnp.float32
  nd = dstf_ref[...] + gdst_ref[...]
  t0 = jnp.cos(bt_ref[...])
  q = (jnp.dot(nd, wqn_ref[...], preferred_element_type=f32)
       + jnp.dot(t0, wqt_ref[...], preferred_element_type=f32)
       + bq_ref[...])
  ns = srcf_ref[...] + gsrc_ref[...]
  te = jnp.cos(delta_ref[...] * wt_ref[...] + bt_ref[...])
  k = (jnp.dot(ns, wkn_ref[...], preferred_element_type=f32)
       + jnp.dot(edge_ref[...], wke_ref[...], preferred_element_type=f32)
       + jnp.dot(te, wkt_ref[...], preferred_element_type=f32)
       + bk_ref[...])
  v = (jnp.dot(ns, wvn_ref[...], preferred_element_type=f32)
       + jnp.dot(edge_ref[...], wve_ref[...], preferred_element_type=f32)
       + jnp.dot(te, wvt_ref[...], preferred_element_type=f32)
       + bv_ref[...])
  scale = f32(1.0 / math.sqrt(DH))
  rows = lax.broadcasted_iota(jnp.int32, (M_BLK, R_BLK), 0)
  cols = lax.broadcasted_iota(jnp.int32, (M_BLK, R_BLK), 1)
  mask = (cols // FANOUT) == rows
  outs = []
  for h in range(N_HEADS):
    qh = q[:, h * DH:(h + 1) * DH] * scale
    kh = k[:, h * DH:(h + 1) * DH]
    vh = v[:, h * DH:(h + 1) * DH]
    s = lax.dot_general(qh, kh, (((1,), (1,)), ((), ())),
                        preferred_element_type=f32)
    s = jnp.where(mask, s, f32(-1e30))
    mx = jnp.max(s, axis=1, keepdims=True)
    e = jnp.exp(s - mx)
    a = e / jnp.sum(e, axis=1, keepdims=True)
    outs.append(lax.dot_general(a, vh, (((1,), (0,)), ((), ())),
                                preferred_element_type=f32))
  out = jnp.concatenate(outs, axis=1)
  proj = jnp.dot(out, wo_ref[...], preferred_element_type=f32) + bo_ref[...]
  h_ref[...] = jnp.maximum(proj + nd, 0.0)


def _gru_body(h_ref, pm_ref, tsrows_ref, lane_ref, mts_ref, ef_ref,
              wtm_ref, btm_ref, wim_ref, wip_ref, wit_ref, wie_ref, bih_ref,
              whh_ref, bhh_ref, nm_ref):
  f32 = jnp.float32
  lane_iota = lax.broadcasted_iota(jnp.int32, (G_BLK, 128), 1)
  sel = (lane_iota == lane_ref[...]).astype(f32)
  pts = jnp.sum(tsrows_ref[...] * sel, axis=1, keepdims=True)
  dt = jnp.maximum(mts_ref[...] - pts, 0.0)
  tf = jnp.cos(dt * wtm_ref[...] + btm_ref[...])
  pm = pm_ref[...]
  gi = (jnp.dot(h_ref[...], wim_ref[...], preferred_element_type=f32)
        + jnp.dot(pm, wip_ref[...], preferred_element_type=f32)
        + jnp.dot(tf, wit_ref[...], preferred_element_type=f32)
        + jnp.dot(ef_ref[...], wie_ref[...], preferred_element_type=f32)
        + bih_ref[...])
  gh = jnp.dot(pm, whh_ref[...], preferred_element_type=f32) + bhh_ref[...]
  i_r, i_z, i_n = gi[:, 0:128], gi[:, 128:256], gi[:, 256:384]
  h_r, h_z, h_n = gh[:, 0:128], gh[:, 128:256], gh[:, 256:384]
  r = jax.nn.sigmoid(i_r + h_r)
  z = jax.nn.sigmoid(i_z + h_z)
  n = jnp.tanh(i_n + r * h_n)
  nm_ref[...] = (1.0 - z) * n + z * pm


def _pred_body(sh_ref, dh_ref, nh_ref, wps_ref, bps_ref, wpd_ref, bpd_ref,
               wpo_ref, bpo_ref, pos_ref, neg_ref):
  f32 = jnp.float32
  sproj = jnp.dot(sh_ref[...], wps_ref[...], preferred_element_type=f32) + bps_ref[...]
  dproj = jnp.dot(dh_ref[...], wpd_ref[...], preferred_element_type=f32) + bpd_ref[...]
  nproj = jnp.dot(nh_ref[...], wpd_ref[...], preferred_element_type=f32) + bpd_ref[...]
  hp = jnp.maximum(sproj + dproj, 0.0)
  hn = jnp.maximum(sproj + nproj, 0.0)
  pos_ref[...] = jnp.dot(hp, wpo_ref[...], preferred_element_type=f32) + bpo_ref[...]
  neg_ref[...] = jnp.dot(hn, wpo_ref[...], preferred_element_type=f32) + bpo_ref[...]


def _row2(x):
  return x.reshape(1, -1)


def kernel(dst_nodes, src_nodes, dst_feat, src_feat, edge_feat_nbr,
           delta_ts_nbr, root_ts, root_edge_feat, memory, memory_ts, params):
  p = params
  f32 = jnp.float32
  n_upd = 2 * B

  dst_nodes = dst_nodes.astype(jnp.int32)
  src_flat = src_nodes.reshape(-1).astype(jnp.int32)
  idx_all = jnp.concatenate([dst_nodes, src_flat], axis=0)
  upd_idx = dst_nodes[:n_upd]

  ts_tab = jnp.pad(memory_ts, (0, TS_ROWS * 128 - N_NODES)).reshape(TS_ROWS, 128)
  ts_row_idx = upd_idx // 128
  ts_lane = (upd_idx % 128).reshape(-1, 1)
  rows, ts_rows = _sc_gather(memory, idx_all, ts_tab, ts_row_idx)

  srcf_flat = src_feat.reshape(GROUP * FANOUT, D_NODE)
  edge_flat = edge_feat_nbr.reshape(GROUP * FANOUT, D_EDGE)
  delta_flat = delta_ts_nbr.reshape(GROUP * FANOUT, 1)

  n_blk = GROUP // M_BLK
  src_off = GROUP // R_BLK

  wq = p["Wq"]
  wk = p["Wk"]
  wv = p["Wv"]
  full = lambda s: pl.BlockSpec(s, lambda i: tuple(0 for _ in s))
  h_total = pl.pallas_call(
      _attn_body,
      grid=(n_blk,),
      in_specs=[
          pl.BlockSpec((M_BLK, D_EMBED), lambda i: (i, 0)),
          pl.BlockSpec((M_BLK, D_NODE), lambda i: (i, 0)),
          pl.BlockSpec((R_BLK, D_EMBED), lambda i: (i + src_off, 0)),
          pl.BlockSpec((R_BLK, D_NODE), lambda i: (i, 0)),
          pl.BlockSpec((R_BLK, D_EDGE), lambda i: (i, 0)),
          pl.BlockSpec((R_BLK, 1), lambda i: (i, 0)),
          full((1, D_TIME)), full((1, D_TIME)),
          full((D_EMBED, D_EMBED)), full((D_TIME, D_EMBED)), full((1, D_EMBED)),
          full((D_EMBED, D_EMBED)), full((D_EDGE, D_EMBED)), full((D_TIME, D_EMBED)), full((1, D_EMBED)),
          full((D_EMBED, D_EMBED)), full((D_EDGE, D_EMBED)), full((D_TIME, D_EMBED)), full((1, D_EMBED)),
          full((D_EMBED, D_EMBED)), full((1, D_EMBED)),
      ],
      out_specs=pl.BlockSpec((M_BLK, D_EMBED), lambda i: (i, 0)),
      out_shape=jax.ShapeDtypeStruct((GROUP, D_EMBED), f32),
  )(
      rows, dst_feat, rows, srcf_flat, edge_flat, delta_flat,
      _row2(p["wt_attn"]), _row2(p["bt_attn"]),
      wq[:D_EMBED], wq[D_EMBED:], _row2(p["bq"]),
      wk[:D_EMBED], wk[D_EMBED:D_EMBED + D_EDGE], wk[D_EMBED + D_EDGE:], _row2(p["bk"]),
      wv[:D_EMBED], wv[D_EMBED:D_EMBED + D_EDGE], wv[D_EMBED + D_EDGE:], _row2(p["bv"]),
      p["Wo"], _row2(p["bo"]),
  )

  mail_ts = jnp.concatenate([root_ts, root_ts], axis=0)
  edge_upd = jnp.concatenate([root_edge_feat, root_edge_feat], axis=0)
  wih_t = p["W_ih"].T
  whh_t = p["W_hh"].T
  d_gin = 3 * D_EMBED

  new_mem = pl.pallas_call(
      _gru_body,
      grid=(n_upd // G_BLK,),
      in_specs=[
          pl.BlockSpec((G_BLK, D_EMBED), lambda i: (i, 0)),
          pl.BlockSpec((G_BLK, D_EMBED), lambda i: (i, 0)),
          pl.BlockSpec((G_BLK, 128), lambda i: (i, 0)),
          pl.BlockSpec((G_BLK, 1), lambda i: (i, 0)),
          pl.BlockSpec((G_BLK, 1), lambda i: (i, 0)),
          pl.BlockSpec((G_BLK, D_EDGE), lambda i: (i, 0)),
          full((1, D_TIME)), full((1, D_TIME)),
          full((D_EMBED, d_gin)), full((D_EMBED, d_gin)),
          full((D_TIME, d_gin)), full((D_EDGE, d_gin)), full((1, d_gin)),
          full((D_EMBED, d_gin)), full((1, d_gin)),
      ],
      out_specs=pl.BlockSpec((G_BLK, D_EMBED), lambda i: (i, 0)),
      out_shape=jax.ShapeDtypeStruct((n_upd, D_EMBED), f32),
  )(
      h_total, rows, ts_rows, ts_lane, mail_ts.reshape(-1, 1), edge_upd,
      _row2(p["wt_mem"]), _row2(p["bt_mem"]),
      wih_t[:D_EMBED], wih_t[D_EMBED:2 * D_EMBED],
      wih_t[2 * D_EMBED:2 * D_EMBED + D_TIME], wih_t[2 * D_EMBED + D_TIME:],
      _row2(p["b_ih"]), whh_t, _row2(p["b_hh"]),
  )

  pos_scores, neg_scores = pl.pallas_call(
      _pred_body,
      grid=(B // P_BLK,),
      in_specs=[
          pl.BlockSpec((P_BLK, D_EMBED), lambda i: (i, 0)),
          pl.BlockSpec((P_BLK, D_EMBED), lambda i: (i + B // P_BLK, 0)),
          pl.BlockSpec((P_BLK, D_EMBED), lambda i: (i + 2 * (B // P_BLK), 0)),
          full((D_EMBED, D_EMBED)), full((1, D_EMBED)),
          full((D_EMBED, D_EMBED)), full((1, D_EMBED)),
          full((D_EMBED, 1)), full((1, 1)),
      ],
      out_specs=[
          pl.BlockSpec((P_BLK, 1), lambda i: (i, 0)),
          pl.BlockSpec((P_BLK, 1), lambda i: (i, 0)),
      ],
      out_shape=[
          jax.ShapeDtypeStruct((B, 1), f32),
          jax.ShapeDtypeStruct((B, 1), f32),
      ],
  )(
      h_total, h_total, h_total,
      p["Wp_src"], _row2(p["bp_src"]), p["Wp_dst"], _row2(p["bp_dst"]),
      p["Wp_out"], _row2(p["bp_out"]),
  )

  order = jnp.zeros((N_NODES,), jnp.int32).at[upd_idx].max(
      jnp.arange(n_upd, dtype=jnp.int32))
  win_src = order[upd_idx]

  mem_ref = jax.new_ref(memory)
  ts_in = jnp.pad(memory_ts, (0, TS_PAD - N_NODES))
  ts_out = _sc_scatter(new_mem, win_src, upd_idx, mail_ts, ts_in, mem_ref)
  new_memory = mem_ref[...]
  new_memory_ts = ts_out[:N_NODES]

  return (pos_scores, neg_scores, new_memory, new_memory_ts)


# speedup vs baseline: 1.5492x; 1.4332x over previous
import dataclasses
import functools
import math

import jax
import jax.numpy as jnp
from jax import lax
from jax.experimental import pallas as pl
from jax.experimental.pallas import tpu as pltpu
from jax.experimental.pallas import tpu_sc as plsc

N_NODES = 100000
B = 4096
GROUP = 3 * B
FANOUT = 16
D_NODE = 128
D_EDGE = 16
D_TIME = 100
D_EMBED = 128
N_HEADS = 2
DH = D_EMBED // N_HEADS

GATHER_WIN = 128
M_BLK = 64
R_BLK = M_BLK * FANOUT
G_BLK = 512
P_BLK = 512
N_WORKERS = 32
TS_ROWS = (N_NODES + 127) // 128
TS_SLICE = 3136
TS_PAD = N_WORKERS * TS_SLICE


_COS_COEFFS = (
    0.9999999997244845, -0.4999999994844787, 0.04166666491096737,
    -0.001388887075225163, 2.4800753857600428e-05, -2.7537477718681674e-07,
    2.062132069269307e-09, -9.768685860223059e-12,
)


def _fast_cos(x):
  f32 = jnp.float32
  n = jnp.floor(x * f32(0.15915494309189535) + f32(0.5))
  r = (x - n * f32(6.28125)) - n * f32(0.0019353071795864769)
  t = r * r
  acc = jnp.full_like(t, f32(_COS_COEFFS[7]))
  for k in range(6, -1, -1):
    acc = acc * t + f32(_COS_COEFFS[k])
  return acc


def _sc_mesh():
  return plsc.VectorSubcoreMesh(core_axis_name="core", subcore_axis_name="subcore")


def _sc_gather(memory, idx_all, ts_tab, ts_row_idx):
  k_rows = idx_all.shape[0]
  k_ts = ts_row_idx.shape[0]
  idx2 = idx_all.reshape(1, k_rows)
  tsr2 = ts_row_idx.reshape(1, k_ts)

  @pl.kernel(
      out_type=(
          jax.ShapeDtypeStruct((k_rows, D_EMBED), jnp.float32),
          jax.ShapeDtypeStruct((k_ts, 128), jnp.float32),
      ),
      mesh=_sc_mesh(),
  )
  def gather_kernel(mem_hbm, idx_hbm, ts_hbm, tsr_hbm, rows_hbm, tsrows_hbm):
    def row_body(i_vmem, o_vmem):
      pltpu.sync_copy(mem_hbm.at[i_vmem.at[0]], o_vmem)

    pltpu.emit_pipeline(
        row_body,
        grid=(k_rows // GATHER_WIN,),
        in_specs=[pl.BlockSpec((1, GATHER_WIN), lambda i: (0, i))],
        out_specs=[pl.BlockSpec((GATHER_WIN, D_EMBED), lambda i: (i, 0))],
        core_axis_name=("core", "subcore"),
        dimension_semantics=(pltpu.PARALLEL,),
    )(idx_hbm, rows_hbm)

    def ts_body(i_vmem, o_vmem):
      pltpu.sync_copy(ts_hbm.at[i_vmem.at[0]], o_vmem)

    pltpu.emit_pipeline(
        ts_body,
        grid=(k_ts // GATHER_WIN,),
        in_specs=[pl.BlockSpec((1, GATHER_WIN), lambda i: (0, i))],
        out_specs=[pl.BlockSpec((GATHER_WIN, 128), lambda i: (i, 0))],
        core_axis_name=("core", "subcore"),
        dimension_semantics=(pltpu.PARALLEL,),
    )(tsr_hbm, tsrows_hbm)

  return gather_kernel(memory, idx2, ts_tab, tsr2)


def _sc_scatter(new_mem, win_src, dst_idx, mail_ts, ts_in, mem_ref):
  n_upd = dst_idx.shape[0]
  win2 = win_src.reshape(1, n_upd)
  dst2 = dst_idx.reshape(1, n_upd)

  cp = pltpu.CompilerParams()
  if "needs_layout_passes" in pltpu.CompilerParams.__dataclass_fields__:
    cp = dataclasses.replace(cp, needs_layout_passes=False)

  @pl.kernel(
      out_type=jax.ShapeDtypeStruct((TS_PAD,), jnp.float32),
      mesh=_sc_mesh(),
      compiler_params=cp,
      scratch_types=[
          pltpu.VMEM((GATHER_WIN, D_EMBED), jnp.float32),
          pltpu.VMEM((TS_SLICE,), jnp.float32),
          pltpu.VMEM((n_upd,), jnp.int32),
          pltpu.VMEM((n_upd,), jnp.int32),
          pltpu.VMEM((n_upd,), jnp.float32),
      ],
  )
  def scatter_kernel(nm_hbm, win_hbm, dst_hbm, mts_hbm, tsin_hbm, mem_hbm,
                     tsout_hbm, rows_vmem, tsslice_v, upd_v, win_v, mail_v):
    def body(w_vmem, d_vmem):
      pltpu.sync_copy(nm_hbm.at[w_vmem.at[0]], rows_vmem)
      pltpu.sync_copy(rows_vmem, mem_hbm.at[d_vmem.at[0]])

    pltpu.emit_pipeline(
        body,
        grid=(n_upd // GATHER_WIN,),
        in_specs=[
            pl.BlockSpec((1, GATHER_WIN), lambda i: (0, i)),
            pl.BlockSpec((1, GATHER_WIN), lambda i: (0, i)),
        ],
        out_specs=[],
        core_axis_name=("core", "subcore"),
        dimension_semantics=(pltpu.PARALLEL,),
    )(win_hbm, dst_hbm)

    wid = lax.axis_index("core") * 16 + lax.axis_index("subcore")
    base = wid * TS_SLICE
    pltpu.sync_copy(tsin_hbm.at[pl.ds(base, TS_SLICE)], tsslice_v)
    pltpu.sync_copy(dst_hbm.at[0], upd_v)
    pltpu.sync_copy(win_hbm.at[0], win_v)
    pltpu.sync_copy(mts_hbm, mail_v)

    @pl.loop(0, n_upd // 16)
    def _(c):
      iv = upd_v[pl.ds(c * 16, 16)]
      wv = win_v[pl.ds(c * 16, 16)]
      vals = plsc.load_gather(mail_v, [wv])
      mask = (iv >= base) & (iv < base + TS_SLICE)
      plsc.store_scatter(tsslice_v, [iv - base], vals, mask=mask)

    pltpu.sync_copy(tsslice_v, tsout_hbm.at[pl.ds(base, TS_SLICE)])

  return scatter_kernel(new_mem, win2, dst2, mail_ts, ts_in, mem_ref)


def _attn_body(gdst_ref, dstf_ref, gsrc_ref, srcf_ref, edge_ref, delta_ref,
               wt_ref, bt_ref, wqn_ref, wqt_ref, bq_ref,
               wkn_ref, wke_ref, wkt_ref, bk_ref,
               wvn_ref, wve_ref, wvt_ref, bv_ref,
               wo_ref, bo_ref, h_ref):
  f32 = jnp.float32
  nd = dstf_ref[...] + gdst_ref[...]
  t0 = _fast_cos(bt_ref[...])
  q = (jnp.dot(nd, wqn_ref[...], preferred_element_type=f32)
       + jnp.dot(t0, wqt_ref[...], preferred_element_type=f32)
       + bq_ref[...])
  ns = srcf_ref[...] + gsrc_ref[...]
  te = _fast_cos(delta_ref[...] * wt_ref[...] + bt_ref[...])
  k = (jnp.dot(ns, wkn_ref[...], preferred_element_type=f32)
       + jnp.dot(edge_ref[...], wke_ref[...], preferred_element_type=f32)
       + jnp.dot(te, wkt_ref[...], preferred_element_type=f32)
       + bk_ref[...])
  v = (jnp.dot(ns, wvn_ref[...], preferred_element_type=f32)
       + jnp.dot(edge_ref[...], wve_ref[...], preferred_element_type=f32)
       + jnp.dot(te, wvt_ref[...], preferred_element_type=f32)
       + bv_ref[...])
  scale = f32(1.0 / math.sqrt(DH))
  rows = lax.broadcasted_iota(jnp.int32, (M_BLK, R_BLK), 0)
  cols = lax.broadcasted_iota(jnp.int32, (M_BLK, R_BLK), 1)
  mask = (cols // FANOUT) == rows
  outs = []
  for h in range(N_HEADS):
    qh = q[:, h * DH:(h + 1) * DH] * scale
    kh = k[:, h * DH:(h + 1) * DH]
    vh = v[:, h * DH:(h + 1) * DH]
    s = lax.dot_general(qh, kh, (((1,), (1,)), ((), ())),
                        preferred_element_type=f32)
    s = jnp.where(mask, s, f32(-1e30))
    mx = jnp.max(s, axis=1, keepdims=True)
    e = jnp.exp(s - mx)
    a = e / jnp.sum(e, axis=1, keepdims=True)
    outs.append(lax.dot_general(a, vh, (((1,), (0,)), ((), ())),
                                preferred_element_type=f32))
  out = jnp.concatenate(outs, axis=1)
  proj = jnp.dot(out, wo_ref[...], preferred_element_type=f32) + bo_ref[...]
  h_ref[...] = jnp.maximum(proj + nd, 0.0)


def _gru_body(h_ref, pm_ref, tsrows_ref, lane_ref, mts_ref, ef_ref,
              wtm_ref, btm_ref, wim_ref, wip_ref, wit_ref, wie_ref, bih_ref,
              whh_ref, bhh_ref, nm_ref):
  f32 = jnp.float32
  lane_iota = lax.broadcasted_iota(jnp.int32, (G_BLK, 128), 1)
  sel = (lane_iota == lane_ref[...]).astype(f32)
  pts = jnp.sum(tsrows_ref[...] * sel, axis=1, keepdims=True)
  dt = jnp.maximum(mts_ref[...] - pts, 0.0)
  tf = _fast_cos(dt * wtm_ref[...] + btm_ref[...])
  pm = pm_ref[...]
  gi = (jnp.dot(h_ref[...], wim_ref[...], preferred_element_type=f32)
        + jnp.dot(pm, wip_ref[...], preferred_element_type=f32)
        + jnp.dot(tf, wit_ref[...], preferred_element_type=f32)
        + jnp.dot(ef_ref[...], wie_ref[...], preferred_element_type=f32)
        + bih_ref[...])
  gh = jnp.dot(pm, whh_ref[...], preferred_element_type=f32) + bhh_ref[...]
  i_r, i_z, i_n = gi[:, 0:128], gi[:, 128:256], gi[:, 256:384]
  h_r, h_z, h_n = gh[:, 0:128], gh[:, 128:256], gh[:, 256:384]
  r = jax.nn.sigmoid(i_r + h_r)
  z = jax.nn.sigmoid(i_z + h_z)
  n = jnp.tanh(i_n + r * h_n)
  nm_ref[...] = (1.0 - z) * n + z * pm


def _pred_body(sh_ref, dh_ref, nh_ref, wps_ref, bps_ref, wpd_ref, bpd_ref,
               wpo_ref, bpo_ref, pos_ref, neg_ref):
  f32 = jnp.float32
  sproj = jnp.dot(sh_ref[...], wps_ref[...], preferred_element_type=f32) + bps_ref[...]
  dproj = jnp.dot(dh_ref[...], wpd_ref[...], preferred_element_type=f32) + bpd_ref[...]
  nproj = jnp.dot(nh_ref[...], wpd_ref[...], preferred_element_type=f32) + bpd_ref[...]
  hp = jnp.maximum(sproj + dproj, 0.0)
  hn = jnp.maximum(sproj + nproj, 0.0)
  pos_ref[...] = jnp.dot(hp, wpo_ref[...], preferred_element_type=f32) + bpo_ref[...]
  neg_ref[...] = jnp.dot(hn, wpo_ref[...], preferred_element_type=f32) + bpo_ref[...]


def _row2(x):
  return x.reshape(1, -1)


def kernel(dst_nodes, src_nodes, dst_feat, src_feat, edge_feat_nbr,
           delta_ts_nbr, root_ts, root_edge_feat, memory, memory_ts, params):
  p = params
  f32 = jnp.float32
  n_upd = 2 * B

  dst_nodes = dst_nodes.astype(jnp.int32)
  src_flat = src_nodes.reshape(-1).astype(jnp.int32)
  idx_all = jnp.concatenate([dst_nodes, src_flat], axis=0)
  upd_idx = dst_nodes[:n_upd]

  ts_tab = jnp.pad(memory_ts, (0, TS_ROWS * 128 - N_NODES)).reshape(TS_ROWS, 128)
  ts_row_idx = upd_idx // 128
  ts_lane = (upd_idx % 128).reshape(-1, 1)
  rows, ts_rows = _sc_gather(memory, idx_all, ts_tab, ts_row_idx)

  srcf_flat = src_feat.reshape(GROUP * FANOUT, D_NODE)
  edge_flat = edge_feat_nbr.reshape(GROUP * FANOUT, D_EDGE)
  delta_flat = delta_ts_nbr.reshape(GROUP * FANOUT, 1)

  n_blk = GROUP // M_BLK
  src_off = GROUP // R_BLK

  wq = p["Wq"]
  wk = p["Wk"]
  wv = p["Wv"]
  full = lambda s: pl.BlockSpec(s, lambda i: tuple(0 for _ in s))
  h_total = pl.pallas_call(
      _attn_body,
      grid=(n_blk,),
      in_specs=[
          pl.BlockSpec((M_BLK, D_EMBED), lambda i: (i, 0)),
          pl.BlockSpec((M_BLK, D_NODE), lambda i: (i, 0)),
          pl.BlockSpec((R_BLK, D_EMBED), lambda i: (i + src_off, 0)),
          pl.BlockSpec((R_BLK, D_NODE), lambda i: (i, 0)),
          pl.BlockSpec((R_BLK, D_EDGE), lambda i: (i, 0)),
          pl.BlockSpec((R_BLK, 1), lambda i: (i, 0)),
          full((1, D_TIME)), full((1, D_TIME)),
          full((D_EMBED, D_EMBED)), full((D_TIME, D_EMBED)), full((1, D_EMBED)),
          full((D_EMBED, D_EMBED)), full((D_EDGE, D_EMBED)), full((D_TIME, D_EMBED)), full((1, D_EMBED)),
          full((D_EMBED, D_EMBED)), full((D_EDGE, D_EMBED)), full((D_TIME, D_EMBED)), full((1, D_EMBED)),
          full((D_EMBED, D_EMBED)), full((1, D_EMBED)),
      ],
      out_specs=pl.BlockSpec((M_BLK, D_EMBED), lambda i: (i, 0)),
      out_shape=jax.ShapeDtypeStruct((GROUP, D_EMBED), f32),
  )(
      rows, dst_feat, rows, srcf_flat, edge_flat, delta_flat,
      _row2(p["wt_attn"]), _row2(p["bt_attn"]),
      wq[:D_EMBED], wq[D_EMBED:], _row2(p["bq"]),
      wk[:D_EMBED], wk[D_EMBED:D_EMBED + D_EDGE], wk[D_EMBED + D_EDGE:], _row2(p["bk"]),
      wv[:D_EMBED], wv[D_EMBED:D_EMBED + D_EDGE], wv[D_EMBED + D_EDGE:], _row2(p["bv"]),
      p["Wo"], _row2(p["bo"]),
  )

  mail_ts = jnp.concatenate([root_ts, root_ts], axis=0)
  edge_upd = jnp.concatenate([root_edge_feat, root_edge_feat], axis=0)
  wih_t = p["W_ih"].T
  whh_t = p["W_hh"].T
  d_gin = 3 * D_EMBED

  new_mem = pl.pallas_call(
      _gru_body,
      grid=(n_upd // G_BLK,),
      in_specs=[
          pl.BlockSpec((G_BLK, D_EMBED), lambda i: (i, 0)),
          pl.BlockSpec((G_BLK, D_EMBED), lambda i: (i, 0)),
          pl.BlockSpec((G_BLK, 128), lambda i: (i, 0)),
          pl.BlockSpec((G_BLK, 1), lambda i: (i, 0)),
          pl.BlockSpec((G_BLK, 1), lambda i: (i, 0)),
          pl.BlockSpec((G_BLK, D_EDGE), lambda i: (i, 0)),
          full((1, D_TIME)), full((1, D_TIME)),
          full((D_EMBED, d_gin)), full((D_EMBED, d_gin)),
          full((D_TIME, d_gin)), full((D_EDGE, d_gin)), full((1, d_gin)),
          full((D_EMBED, d_gin)), full((1, d_gin)),
      ],
      out_specs=pl.BlockSpec((G_BLK, D_EMBED), lambda i: (i, 0)),
      out_shape=jax.ShapeDtypeStruct((n_upd, D_EMBED), f32),
  )(
      h_total, rows, ts_rows, ts_lane, mail_ts.reshape(-1, 1), edge_upd,
      _row2(p["wt_mem"]), _row2(p["bt_mem"]),
      wih_t[:D_EMBED], wih_t[D_EMBED:2 * D_EMBED],
      wih_t[2 * D_EMBED:2 * D_EMBED + D_TIME], wih_t[2 * D_EMBED + D_TIME:],
      _row2(p["b_ih"]), whh_t, _row2(p["b_hh"]),
  )

  pos_scores, neg_scores = pl.pallas_call(
      _pred_body,
      grid=(B // P_BLK,),
      in_specs=[
          pl.BlockSpec((P_BLK, D_EMBED), lambda i: (i, 0)),
          pl.BlockSpec((P_BLK, D_EMBED), lambda i: (i + B // P_BLK, 0)),
          pl.BlockSpec((P_BLK, D_EMBED), lambda i: (i + 2 * (B // P_BLK), 0)),
          full((D_EMBED, D_EMBED)), full((1, D_EMBED)),
          full((D_EMBED, D_EMBED)), full((1, D_EMBED)),
          full((D_EMBED, 1)), full((1, 1)),
      ],
      out_specs=[
          pl.BlockSpec((P_BLK, 1), lambda i: (i, 0)),
          pl.BlockSpec((P_BLK, 1), lambda i: (i, 0)),
      ],
      out_shape=[
          jax.ShapeDtypeStruct((B, 1), f32),
          jax.ShapeDtypeStruct((B, 1), f32),
      ],
  )(
      h_total, h_total, h_total,
      p["Wp_src"], _row2(p["bp_src"]), p["Wp_dst"], _row2(p["bp_dst"]),
      p["Wp_out"], _row2(p["bp_out"]),
  )

  order = jnp.zeros((N_NODES,), jnp.int32).at[upd_idx].max(
      jnp.arange(n_upd, dtype=jnp.int32))
  win_src = order[upd_idx]

  mem_ref = jax.new_ref(memory)
  ts_in = jnp.pad(memory_ts, (0, TS_PAD - N_NODES))
  ts_out = _sc_scatter(new_mem, win_src, upd_idx, mail_ts, ts_in, mem_ref)
  new_memory = mem_ref[...]
  new_memory_ts = ts_out[:N_NODES]

  return (pos_scores, neg_scores, new_memory, new_memory_ts)
